# baseline (speedup 1.0000x reference)
import jax
import jax.numpy as jnp
from jax.experimental import pallas as pl
from jax.experimental.pallas import tpu as pltpu
from jax.experimental.pallas import tpu_sc as plsc

_WINDOW = 128


def kernel(token_ids, weight):
    b, t = token_ids.shape
    n = b * t
    d = weight.shape[1]
    idx = token_ids.reshape(1, n).astype(jnp.int32)

    mesh = plsc.VectorSubcoreMesh(core_axis_name="core",
                                  subcore_axis_name="subcore")

    @pl.kernel(out_type=jax.ShapeDtypeStruct((n, d), weight.dtype), mesh=mesh,
               compiler_params=pltpu.CompilerParams(use_tc_tiling_on_sc=False))
    def gather_kernel(table_hbm, idx_hbm, out_hbm):
        def body(idx_vmem, out_vmem):
            pltpu.sync_copy(table_hbm.at[idx_vmem.at[0]], out_vmem)

        pltpu.emit_pipeline(
            body,
            grid=(n // _WINDOW,),
            in_specs=[pl.BlockSpec((1, _WINDOW), lambda i: (0, i))],
            out_specs=[pl.BlockSpec((_WINDOW, d), lambda i: (i, 0))],
            core_axis_name=("core", "subcore"),
            dimension_semantics=(pltpu.PARALLEL,),
        )(idx_hbm, out_hbm)

    return gather_kernel(weight, idx).reshape(b, t, d)


# speedup vs baseline: 1.0708x; 1.0708x over previous
import jax
import jax.numpy as jnp
from jax.experimental import pallas as pl
from jax.experimental.pallas import tpu as pltpu
from jax.experimental.pallas import tpu_sc as plsc

_WINDOW = 512


def kernel(token_ids, weight):
    b, t = token_ids.shape
    n = b * t
    d = weight.shape[1]
    idx = token_ids.reshape(1, n).astype(jnp.int32)

    mesh = plsc.VectorSubcoreMesh(core_axis_name="core",
                                  subcore_axis_name="subcore")

    @pl.kernel(out_type=jax.ShapeDtypeStruct((n, d), weight.dtype), mesh=mesh,
               compiler_params=pltpu.CompilerParams(use_tc_tiling_on_sc=False))
    def gather_kernel(table_hbm, idx_hbm, out_hbm):
        def body(idx_vmem, out_vmem):
            pltpu.sync_copy(table_hbm.at[idx_vmem.at[0]], out_vmem)

        pltpu.emit_pipeline(
            body,
            grid=(n // _WINDOW,),
            in_specs=[pl.BlockSpec((1, _WINDOW), lambda i: (0, i))],
            out_specs=[pl.BlockSpec((_WINDOW, d), lambda i: (i, 0))],
            core_axis_name=("core", "subcore"),
            dimension_semantics=(pltpu.PARALLEL,),
        )(idx_hbm, out_hbm)

    return gather_kernel(weight, idx).reshape(b, t, d)


# speedup vs baseline: 1.0771x; 1.0058x over previous
import jax
import jax.numpy as jnp
from jax import lax
from jax.experimental import pallas as pl
from jax.experimental.pallas import tpu as pltpu
from jax.experimental.pallas import tpu_sc as plsc

_W = 256
_NBUF = 4
_NW = 32


def kernel(token_ids, weight):
    b, t = token_ids.shape
    n = b * t
    d = weight.shape[1]
    idx = token_ids.reshape(n).astype(jnp.int32)
    n_per = n // _NW
    nchunk = n_per // _W
    assert n_per % _W == 0 and nchunk % _NBUF == 0

    mesh = plsc.VectorSubcoreMesh(core_axis_name="core",
                                  subcore_axis_name="subcore")

    scratch = ([pltpu.VMEM((n_per,), jnp.int32)]
               + [pltpu.VMEM((_W, d), jnp.float32) for _ in range(_NBUF)]
               + [pltpu.SemaphoreType.DMA for _ in range(2 * _NBUF)])

    @pl.kernel(out_type=jax.ShapeDtypeStruct((n, d), weight.dtype), mesh=mesh,
               compiler_params=pltpu.CompilerParams(use_tc_tiling_on_sc=False),
               scratch_types=scratch)
    def gather_kernel(table_hbm, idx_hbm, out_hbm, idx_v, *rest):
        bufs = rest[:_NBUF]
        gsem = rest[_NBUF:2 * _NBUF]
        wsem = rest[2 * _NBUF:]
        wid = lax.axis_index("subcore") * 2 + lax.axis_index("core")
        base = wid * n_per
        pltpu.sync_copy(idx_hbm.at[pl.ds(base, n_per)], idx_v)

        def start_gather(bi, chunk):
            pltpu.make_async_copy(
                table_hbm.at[idx_v.at[pl.ds(chunk * _W, _W)]],
                bufs[bi], gsem[bi]).start()

        def wait_gather(bi):
            pltpu.make_async_copy(
                table_hbm.at[idx_v.at[pl.ds(0, _W)]],
                bufs[bi], gsem[bi]).wait()

        def start_wb(bi, chunk):
            pltpu.make_async_copy(
                bufs[bi], out_hbm.at[pl.ds(base + chunk * _W, _W)],
                wsem[bi]).start()

        def wait_wb(bi):
            pltpu.make_async_copy(
                bufs[bi], out_hbm.at[pl.ds(base, _W)], wsem[bi]).wait()

        for bi in range(_NBUF):
            start_gather(bi, bi)

        @pl.loop(0, nchunk - _NBUF, step=_NBUF)
        def _(g):
            for bi in range(_NBUF):
                chunk = g + bi
                wait_gather(bi)
                start_wb(bi, chunk)
                wait_wb(bi)
                start_gather(bi, chunk + _NBUF)

        for bi in range(_NBUF):
            wait_gather(bi)
            start_wb(bi, nchunk - _NBUF + bi)
        for bi in range(_NBUF):
            wait_wb(bi)

    return gather_kernel(weight, idx).reshape(b, t, d)
